# baseline (speedup 1.0000x reference)
import functools

import jax
import jax.numpy as jnp
from jax import lax
from jax.experimental import pallas as pl
from jax.experimental.pallas import tpu as pltpu
from jax.experimental.pallas import tpu_sc as plsc

_NC = 2
_NS = 16
_K = 40
_NBUF = 5

_H = 256
_HH = 128
_BN = 1000


def _half_spec():
    return pl.BlockSpec((_BN, _HH), lambda i: (i, 0))


def _full_spec(shape):
    return pl.BlockSpec(shape, lambda i: (0, 0))


def _enc_body(x_ref, w1_ref, b1_ref, w2_ref, b2_ref, g_ref,
              h_ref, hw0_ref, hw1_ref):
    bf = jnp.bfloat16
    h1 = jnp.maximum(
        jnp.dot(x_ref[...].astype(bf), w1_ref[...],
                preferred_element_type=jnp.float32) + b1_ref[...], 0.0)
    h2 = jnp.dot(h1.astype(bf), w2_ref[...],
                 preferred_element_type=jnp.float32) + b2_ref[...]
    h_ref[...] = h2
    hw = jnp.dot(h2.astype(bf), g_ref[...], preferred_element_type=jnp.float32)
    hw0_ref[...] = hw[:, :_HH]
    hw1_ref[...] = hw[:, _HH:]


def _gru_gates(m0_ref, m1_ref, h_ref, wih_ref, whh_ref, bih_ref, bhh_ref):
    bf = jnp.bfloat16
    m = jnp.concatenate([m0_ref[...], m1_ref[...]], axis=1).astype(bf)
    h = h_ref[...]
    gi = lax.dot_general(m, wih_ref[...], (((1,), (1,)), ((), ())),
                         preferred_element_type=jnp.float32) + bih_ref[...]
    gh = lax.dot_general(h.astype(bf), whh_ref[...], (((1,), (1,)), ((), ())),
                         preferred_element_type=jnp.float32) + bhh_ref[...]
    r = jax.nn.sigmoid(gi[:, :_H] + gh[:, :_H])
    z = jax.nn.sigmoid(gi[:, _H:2 * _H] + gh[:, _H:2 * _H])
    n = jnp.tanh(gi[:, 2 * _H:] + r * gh[:, 2 * _H:])
    return (1.0 - z) * n + z * h


def _gru_mid_body(m0_ref, m1_ref, h_ref, wih_ref, whh_ref, bih_ref, bhh_ref,
                  g_ref, hout_ref, hw0_ref, hw1_ref):
    hn = _gru_gates(m0_ref, m1_ref, h_ref, wih_ref, whh_ref, bih_ref, bhh_ref)
    hout_ref[...] = hn
    hw = jnp.dot(hn.astype(jnp.bfloat16), g_ref[...],
                 preferred_element_type=jnp.float32)
    hw0_ref[...] = hw[:, :_HH]
    hw1_ref[...] = hw[:, _HH:]


def _gru_final_body(m0_ref, m1_ref, h_ref, wih_ref, whh_ref, bih_ref, bhh_ref,
                    dw1_ref, db1_ref, dw2_ref, db2_ref, out_ref):
    hn = _gru_gates(m0_ref, m1_ref, h_ref, wih_ref, whh_ref, bih_ref, bhh_ref)
    y = jnp.maximum(hn, 0.0)
    o = jnp.maximum(
        jnp.dot(y.astype(jnp.bfloat16), dw1_ref[...],
                preferred_element_type=jnp.float32) + db1_ref[...], 0.0)
    o2 = jnp.dot(o.astype(jnp.bfloat16), dw2_ref[...],
                 preferred_element_type=jnp.float32) + db2_ref[...]
    out_ref[...] = jax.nn.sigmoid(o2)


_SUP = 25


def _make_segsum(n_nodes, n_edges):
    ep = n_edges // _NS
    nch = ep // _K
    nsup = nch // _SUP
    rz = (n_nodes // _NS) // 8 * 8
    rrem = n_nodes - rz * _NS
    mesh = plsc.VectorSubcoreMesh(core_axis_name="c", subcore_axis_name="s",
                                  num_cores=_NC, num_subcores=_NS)

    @functools.partial(
        pl.kernel,
        out_type=jax.ShapeDtypeStruct((_NC, n_nodes, _HH), jnp.float32),
        mesh=mesh,
        scratch_types=[
            pltpu.VMEM((_SUP, _K), jnp.int32),
            pltpu.VMEM((_SUP, _K), jnp.int32),
            pltpu.VMEM((_SUP, _K), jnp.int32),
            pltpu.VMEM((_SUP, _K), jnp.int32),
        ]
        + [pltpu.VMEM((_K, _HH), jnp.float32) for _ in range(_NBUF)]
        + [pltpu.VMEM_SHARED((n_nodes, _HH), jnp.float32)]
        + [pltpu.SemaphoreType.DMA for _ in range(2 * _NBUF + 2)],
    )
    def segsum(hw, src4, dst4, zblk, m, *refs):
        src0, dst0, src1, dst1 = refs[0:4]
        rowsb = refs[4:4 + _NBUF]
        acc = refs[4 + _NBUF]
        sems = refs[5 + _NBUF:5 + 2 * _NBUF]
        ssems = refs[5 + 2 * _NBUF:5 + 3 * _NBUF]
        isems = refs[5 + 3 * _NBUF:5 + 3 * _NBUF + 2]
        c = lax.axis_index("c")
        s = lax.axis_index("s")
        srcb = (src0, src1)
        dstb = (dst0, dst1)
        srow = (c * _NS + s) * nsup
        drow = s * nsup
        nt = nsup // 2
        two_sup = 2 * _SUP

        pltpu.sync_copy(zblk, acc.at[pl.ds(s * rz, rz)])
        if rrem:
            @pl.when(s == _NS - 1)
            def _():
                pltpu.sync_copy(zblk.at[pl.ds(0, rrem)],
                                acc.at[pl.ds(_NS * rz, rrem)])
        plsc.subcore_barrier()

        def load_idx(u, p):
            pltpu.async_copy(src4.at[srow + u], srcb[p], isems[p])
            pltpu.async_copy(dst4.at[drow + u], dstb[p], isems[p])

        def wait_idx(u, p):
            pltpu.make_async_copy(src4.at[srow + u], srcb[p], isems[p]).wait()
            pltpu.make_async_copy(dst4.at[drow + u], dstb[p], isems[p]).wait()

        def issue_gather(jj, b):
            pp, jx = divmod(jj, _SUP)
            pltpu.async_copy(hw.at[srcb[pp].at[jx]], rowsb[b], sems[b])

        def wait_scatter(b, pp, jx):
            pltpu.make_async_copy(rowsb[b], acc.at[dstb[pp].at[jx]],
                                  ssems[b]).wait()

        load_idx(0, 0)
        wait_idx(0, 0)
        for b in range(2):
            issue_gather(b, b)

        def sup_pair(t, carry):
            for jj in range(two_sup):
                pp, jx = divmod(jj, _SUP)
                b = jj % _NBUF
                if jj == 3:
                    load_idx(2 * t + 1, 1)
                if jj == 23:
                    wait_idx(2 * t + 1, 1)
                if jj == 28:
                    @pl.when(t + 1 < nt)
                    def _():
                        load_idx(2 * t + 2, 0)
                if jj == 48:
                    @pl.when(t + 1 < nt)
                    def _():
                        wait_idx(2 * t + 2, 0)
                pltpu.make_async_copy(hw.at[srcb[pp].at[jx]], rowsb[b],
                                      sems[b]).wait()
                pltpu.async_copy(rowsb[b], acc.at[dstb[pp].at[jx]],
                                 ssems[b], add=True)
                b2 = (jj + 2) % _NBUF
                if jj < 3:
                    @pl.when(t > 0)
                    def _():
                        wait_scatter(b2, pp, jx)
                else:
                    wait_scatter(b2, pp, jx)
                if jj + 2 < two_sup:
                    issue_gather(jj + 2, b2)
                else:
                    @pl.when(t + 1 < nt)
                    def _():
                        issue_gather(jj + 2 - two_sup, b2)
            return carry

        lax.fori_loop(0, nt, sup_pair, 0)
        for jj in range(two_sup - 3, two_sup):
            pp, jx = divmod(jj, _SUP)
            wait_scatter(jj % _NBUF, pp, jx)
        plsc.subcore_barrier()
        pltpu.sync_copy(acc.at[pl.ds(s * rz, rz)],
                        m.at[c, pl.ds(s * rz, rz)])
        if rrem:
            @pl.when(s == _NS - 1)
            def _():
                pltpu.sync_copy(acc.at[pl.ds(_NS * rz, rrem)],
                                m.at[c, pl.ds(_NS * rz, rrem)])

    return segsum


def kernel(x, edge_index, enc_w1, enc_b1, enc_w2, enc_b2, ggc_w,
           w_ih, w_hh, b_ih, b_hh, dec_w1, dec_b1, dec_w2, dec_b2):
    n_nodes, d_in = x.shape
    h_dim = enc_w1.shape[1]
    n_layers = ggc_w.shape[0]
    n_edges = edge_index.shape[1]
    grid = (n_nodes // _BN,)

    nsup = (n_edges // _NS) // _K // _SUP
    src_half = edge_index[0].reshape(_NS * nsup, _SUP, _K)
    src2 = jnp.concatenate([src_half, src_half + n_nodes], axis=0)
    dst2 = edge_index[1].reshape(_NS * nsup, _SUP, _K)
    zblk = jnp.zeros(((n_nodes // _NS) // 8 * 8, _HH), jnp.float32)
    b1 = enc_b1.reshape(1, h_dim)
    b2 = enc_b2.reshape(1, h_dim)
    bih = b_ih.reshape(1, 3 * h_dim)
    bhh = b_hh.reshape(1, 3 * h_dim)
    db1 = dec_b1.reshape(1, h_dim)
    db2 = dec_b2.reshape(1, d_in)
    bf = jnp.bfloat16
    enc_w1 = enc_w1.astype(bf)
    enc_w2 = enc_w2.astype(bf)
    ggc_w = ggc_w.astype(bf)
    w_ih = w_ih.astype(bf)
    w_hh = w_hh.astype(bf)
    dec_w1 = dec_w1.astype(bf)
    dec_w2 = dec_w2.astype(bf)

    enc = pl.pallas_call(
        _enc_body,
        grid=grid,
        in_specs=[
            pl.BlockSpec((_BN, d_in), lambda i: (i, 0)),
            _full_spec((d_in, h_dim)),
            _full_spec((1, h_dim)),
            _full_spec((h_dim, h_dim)),
            _full_spec((1, h_dim)),
            _full_spec((h_dim, h_dim)),
        ],
        out_specs=[
            pl.BlockSpec((_BN, h_dim), lambda i: (i, 0)),
            _half_spec(), _half_spec(),
        ],
        out_shape=[
            jax.ShapeDtypeStruct((n_nodes, h_dim), jnp.float32),
            jax.ShapeDtypeStruct((n_nodes, _HH), jnp.float32),
            jax.ShapeDtypeStruct((n_nodes, _HH), jnp.float32),
        ],
    )
    h, hw0, hw1 = enc(x, enc_w1, b1, enc_w2, b2, ggc_w[0])

    segsum = _make_segsum(n_nodes, n_edges)

    nb = n_nodes // _BN
    gru_common_specs = [
        pl.BlockSpec((_BN, _HH), lambda i: (i, 0)),
        pl.BlockSpec((_BN, _HH), lambda i: (i + nb, 0)),
        pl.BlockSpec((_BN, h_dim), lambda i: (i, 0)),
        _full_spec((3 * h_dim, h_dim)),
        _full_spec((3 * h_dim, h_dim)),
        _full_spec((1, 3 * h_dim)),
        _full_spec((1, 3 * h_dim)),
    ]
    gru_mid = pl.pallas_call(
        _gru_mid_body,
        grid=grid,
        in_specs=gru_common_specs + [_full_spec((h_dim, h_dim))],
        out_specs=[
            pl.BlockSpec((_BN, h_dim), lambda i: (i, 0)),
            _half_spec(), _half_spec(),
        ],
        out_shape=[
            jax.ShapeDtypeStruct((n_nodes, h_dim), jnp.float32),
            jax.ShapeDtypeStruct((n_nodes, _HH), jnp.float32),
            jax.ShapeDtypeStruct((n_nodes, _HH), jnp.float32),
        ],
    )
    gru_final = pl.pallas_call(
        _gru_final_body,
        grid=grid,
        in_specs=gru_common_specs + [
            _full_spec((h_dim, h_dim)),
            _full_spec((1, h_dim)),
            _full_spec((h_dim, d_in)),
            _full_spec((1, d_in)),
        ],
        out_specs=[pl.BlockSpec((_BN, d_in), lambda i: (i, 0))],
        out_shape=[jax.ShapeDtypeStruct((n_nodes, d_in), jnp.float32)],
    )

    for l in range(n_layers):
        hw_st = jnp.concatenate([hw0, hw1], axis=0)
        m = segsum(hw_st, src2, dst2, zblk)
        m = m.reshape(_NC * n_nodes, _HH)
        if l + 1 < n_layers:
            h, hw0, hw1 = gru_mid(m, m, h, w_ih, w_hh, bih, bhh, ggc_w[l + 1])
        else:
            (out,) = gru_final(m, m, h, w_ih, w_hh, bih, bhh,
                               dec_w1, db1, dec_w2, db2)
    return out


# speedup vs baseline: 1.2191x; 1.2191x over previous
import functools

import jax
import jax.numpy as jnp
from jax import lax
from jax.experimental import pallas as pl
from jax.experimental.pallas import tpu as pltpu
from jax.experimental.pallas import tpu_sc as plsc

_NC = 2
_NS = 16
_K = 100

_H = 256
_HH = 128
_BN = 1000


def _half_spec():
    return pl.BlockSpec((_BN, _HH), lambda i: (i, 0))


def _full_spec(shape):
    return pl.BlockSpec(shape, lambda i: (0, 0))


def _enc_body(x_ref, w1_ref, b1_ref, w2_ref, b2_ref, g_ref,
              h_ref, hw0_ref, hw1_ref):
    bf = jnp.bfloat16
    h1 = jnp.maximum(
        jnp.dot(x_ref[...].astype(bf), w1_ref[...],
                preferred_element_type=jnp.float32) + b1_ref[...], 0.0)
    h2 = jnp.dot(h1.astype(bf), w2_ref[...],
                 preferred_element_type=jnp.float32) + b2_ref[...]
    h_ref[...] = h2
    hw = jnp.dot(h2.astype(bf), g_ref[...], preferred_element_type=jnp.float32)
    hw0_ref[...] = hw[:, :_HH]
    hw1_ref[...] = hw[:, _HH:]


def _gru_gates(m0_ref, m1_ref, h_ref, wih_ref, whh_ref, bih_ref, bhh_ref):
    bf = jnp.bfloat16
    m = jnp.concatenate([m0_ref[...], m1_ref[...]], axis=1).astype(bf)
    h = h_ref[...]
    gi = lax.dot_general(m, wih_ref[...], (((1,), (1,)), ((), ())),
                         preferred_element_type=jnp.float32) + bih_ref[...]
    gh = lax.dot_general(h.astype(bf), whh_ref[...], (((1,), (1,)), ((), ())),
                         preferred_element_type=jnp.float32) + bhh_ref[...]
    r = jax.nn.sigmoid(gi[:, :_H] + gh[:, :_H])
    z = jax.nn.sigmoid(gi[:, _H:2 * _H] + gh[:, _H:2 * _H])
    n = jnp.tanh(gi[:, 2 * _H:] + r * gh[:, 2 * _H:])
    return (1.0 - z) * n + z * h


def _gru_mid_body(m0_ref, m1_ref, h_ref, wih_ref, whh_ref, bih_ref, bhh_ref,
                  g_ref, hout_ref, hw0_ref, hw1_ref):
    hn = _gru_gates(m0_ref, m1_ref, h_ref, wih_ref, whh_ref, bih_ref, bhh_ref)
    hout_ref[...] = hn
    hw = jnp.dot(hn.astype(jnp.bfloat16), g_ref[...],
                 preferred_element_type=jnp.float32)
    hw0_ref[...] = hw[:, :_HH]
    hw1_ref[...] = hw[:, _HH:]


def _gru_final_body(m0_ref, m1_ref, h_ref, wih_ref, whh_ref, bih_ref, bhh_ref,
                    dw1_ref, db1_ref, dw2_ref, db2_ref, out_ref):
    hn = _gru_gates(m0_ref, m1_ref, h_ref, wih_ref, whh_ref, bih_ref, bhh_ref)
    y = jnp.maximum(hn, 0.0)
    o = jnp.maximum(
        jnp.dot(y.astype(jnp.bfloat16), dw1_ref[...],
                preferred_element_type=jnp.float32) + db1_ref[...], 0.0)
    o2 = jnp.dot(o.astype(jnp.bfloat16), dw2_ref[...],
                 preferred_element_type=jnp.float32) + db2_ref[...]
    out_ref[...] = jax.nn.sigmoid(o2)


_SUP = 25


def _make_segsum(n_nodes, n_edges):
    ep = n_edges // _NS
    nch = ep // _K
    nsup = nch // _SUP
    rz = (n_nodes // _NS) // 8 * 8
    rrem = n_nodes - rz * _NS
    mesh = plsc.VectorSubcoreMesh(core_axis_name="c", subcore_axis_name="s",
                                  num_cores=_NC, num_subcores=_NS)

    @functools.partial(
        pl.kernel,
        out_type=jax.ShapeDtypeStruct((_NC, n_nodes, _HH), jnp.float32),
        mesh=mesh,
        scratch_types=[
            pltpu.VMEM((_SUP, _K), jnp.int32),
            pltpu.VMEM((_SUP, _K), jnp.int32),
            pltpu.VMEM((_SUP, _K), jnp.int32),
            pltpu.VMEM((_SUP, _K), jnp.int32),
            pltpu.VMEM((_K, _HH), jnp.float32),
            pltpu.VMEM((_K, _HH), jnp.float32),
            pltpu.VMEM_SHARED((n_nodes, _HH), jnp.float32),
            pltpu.SemaphoreType.DMA,
            pltpu.SemaphoreType.DMA,
            pltpu.SemaphoreType.DMA,
            pltpu.SemaphoreType.DMA,
            pltpu.SemaphoreType.DMA,
            pltpu.SemaphoreType.DMA,
        ],
    )
    def segsum(hw, src4, dst4, zblk, m,
               src0, dst0, src1, dst1, rows0, rows1, acc,
               sem0, sem1, ssem0, ssem1, isem0, isem1):
        c = lax.axis_index("c")
        s = lax.axis_index("s")
        rowsb = (rows0, rows1)
        sems = (sem0, sem1)
        srcb = (src0, src1)
        dstb = (dst0, dst1)
        isems = (isem0, isem1)
        srow = (c * _NS + s) * nsup
        drow = s * nsup

        pltpu.sync_copy(zblk, acc.at[pl.ds(s * rz, rz)])
        if rrem:
            @pl.when(s == _NS - 1)
            def _():
                pltpu.sync_copy(zblk.at[pl.ds(0, rrem)],
                                acc.at[pl.ds(_NS * rz, rrem)])
        plsc.subcore_barrier()

        def load_idx(u, p):
            pltpu.async_copy(src4.at[srow + u], srcb[p], isems[p])
            pltpu.async_copy(dst4.at[drow + u], dstb[p], isems[p])

        def wait_idx(u, p):
            pltpu.make_async_copy(src4.at[srow + u], srcb[p], isems[p]).wait()
            pltpu.make_async_copy(dst4.at[drow + u], dstb[p], isems[p]).wait()

        def issue_gather(p, j, b):
            pltpu.async_copy(hw.at[srcb[p].at[j]], rowsb[b], sems[b])

        def wait_gather(p, j, b):
            pltpu.make_async_copy(hw.at[srcb[p].at[j]], rowsb[b], sems[b]).wait()

        load_idx(0, 0)
        load_idx(1, 1)
        wait_idx(0, 0)
        for b in range(2):
            issue_gather(0, b, b)

        def sup_pair(t, carry):
            for p in range(2):
                u = 2 * t + p
                q = 1 - p
                @pl.when(u + 1 < nsup)
                def _():
                    wait_idx(u + 1, q)
                for j in range(_SUP):
                    b = (p + j) % 2
                    wait_gather(p, j, b)
                    pltpu.sync_copy(rowsb[b], acc.at[dstb[p].at[j]], add=True)
                    if j + 2 < _SUP:
                        issue_gather(p, j + 2, b)
                    else:
                        @pl.when(u + 1 < nsup)
                        def _():
                            issue_gather(q, j + 2 - _SUP, b)
                @pl.when(u + 2 < nsup)
                def _():
                    load_idx(u + 2, p)
            return carry

        lax.fori_loop(0, nsup // 2, sup_pair, 0)
        plsc.subcore_barrier()
        pltpu.sync_copy(acc.at[pl.ds(s * rz, rz)],
                        m.at[c, pl.ds(s * rz, rz)])
        if rrem:
            @pl.when(s == _NS - 1)
            def _():
                pltpu.sync_copy(acc.at[pl.ds(_NS * rz, rrem)],
                                m.at[c, pl.ds(_NS * rz, rrem)])

    return segsum


def kernel(x, edge_index, enc_w1, enc_b1, enc_w2, enc_b2, ggc_w,
           w_ih, w_hh, b_ih, b_hh, dec_w1, dec_b1, dec_w2, dec_b2):
    n_nodes, d_in = x.shape
    h_dim = enc_w1.shape[1]
    n_layers = ggc_w.shape[0]
    n_edges = edge_index.shape[1]
    grid = (n_nodes // _BN,)

    nsup = (n_edges // _NS) // _K // _SUP
    src_half = edge_index[0].reshape(_NS * nsup, _SUP, _K)
    src2 = jnp.concatenate([src_half, src_half + n_nodes], axis=0)
    dst2 = edge_index[1].reshape(_NS * nsup, _SUP, _K)
    zblk = jnp.zeros(((n_nodes // _NS) // 8 * 8, _HH), jnp.float32)
    b1 = enc_b1.reshape(1, h_dim)
    b2 = enc_b2.reshape(1, h_dim)
    bih = b_ih.reshape(1, 3 * h_dim)
    bhh = b_hh.reshape(1, 3 * h_dim)
    db1 = dec_b1.reshape(1, h_dim)
    db2 = dec_b2.reshape(1, d_in)
    bf = jnp.bfloat16
    enc_w1 = enc_w1.astype(bf)
    enc_w2 = enc_w2.astype(bf)
    ggc_w = ggc_w.astype(bf)
    w_ih = w_ih.astype(bf)
    w_hh = w_hh.astype(bf)
    dec_w1 = dec_w1.astype(bf)
    dec_w2 = dec_w2.astype(bf)

    enc = pl.pallas_call(
        _enc_body,
        grid=grid,
        in_specs=[
            pl.BlockSpec((_BN, d_in), lambda i: (i, 0)),
            _full_spec((d_in, h_dim)),
            _full_spec((1, h_dim)),
            _full_spec((h_dim, h_dim)),
            _full_spec((1, h_dim)),
            _full_spec((h_dim, h_dim)),
        ],
        out_specs=[
            pl.BlockSpec((_BN, h_dim), lambda i: (i, 0)),
            _half_spec(), _half_spec(),
        ],
        out_shape=[
            jax.ShapeDtypeStruct((n_nodes, h_dim), jnp.float32),
            jax.ShapeDtypeStruct((n_nodes, _HH), jnp.float32),
            jax.ShapeDtypeStruct((n_nodes, _HH), jnp.float32),
        ],
    )
    h, hw0, hw1 = enc(x, enc_w1, b1, enc_w2, b2, ggc_w[0])

    segsum = _make_segsum(n_nodes, n_edges)

    nb = n_nodes // _BN
    gru_common_specs = [
        pl.BlockSpec((_BN, _HH), lambda i: (i, 0)),
        pl.BlockSpec((_BN, _HH), lambda i: (i + nb, 0)),
        pl.BlockSpec((_BN, h_dim), lambda i: (i, 0)),
        _full_spec((3 * h_dim, h_dim)),
        _full_spec((3 * h_dim, h_dim)),
        _full_spec((1, 3 * h_dim)),
        _full_spec((1, 3 * h_dim)),
    ]
    gru_mid = pl.pallas_call(
        _gru_mid_body,
        grid=grid,
        in_specs=gru_common_specs + [_full_spec((h_dim, h_dim))],
        out_specs=[
            pl.BlockSpec((_BN, h_dim), lambda i: (i, 0)),
            _half_spec(), _half_spec(),
        ],
        out_shape=[
            jax.ShapeDtypeStruct((n_nodes, h_dim), jnp.float32),
            jax.ShapeDtypeStruct((n_nodes, _HH), jnp.float32),
            jax.ShapeDtypeStruct((n_nodes, _HH), jnp.float32),
        ],
    )
    gru_final = pl.pallas_call(
        _gru_final_body,
        grid=grid,
        in_specs=gru_common_specs + [
            _full_spec((h_dim, h_dim)),
            _full_spec((1, h_dim)),
            _full_spec((h_dim, d_in)),
            _full_spec((1, d_in)),
        ],
        out_specs=[pl.BlockSpec((_BN, d_in), lambda i: (i, 0))],
        out_shape=[jax.ShapeDtypeStruct((n_nodes, d_in), jnp.float32)],
    )

    for l in range(n_layers):
        hw_st = jnp.concatenate([hw0, hw1], axis=0)
        m = segsum(hw_st, src2, dst2, zblk)
        m = m.reshape(_NC * n_nodes, _HH)
        if l + 1 < n_layers:
            h, hw0, hw1 = gru_mid(m, m, h, w_ih, w_hh, bih, bhh, ggc_w[l + 1])
        else:
            (out,) = gru_final(m, m, h, w_ih, w_hh, bih, bhh,
                               dec_w1, db1, dec_w2, db2)
    return out


# speedup vs baseline: 1.2745x; 1.0454x over previous
import functools

import jax
import jax.numpy as jnp
from jax import lax
from jax.experimental import pallas as pl
from jax.experimental.pallas import tpu as pltpu
from jax.experimental.pallas import tpu_sc as plsc

_NC = 2
_NS = 16
_K = 125

_H = 256
_HH = 128
_BN = 1000


def _half_spec():
    return pl.BlockSpec((_BN, _HH), lambda i: (i, 0))


def _full_spec(shape):
    return pl.BlockSpec(shape, lambda i: (0, 0))


def _enc_body(x_ref, w1_ref, b1_ref, w2_ref, b2_ref, g_ref,
              h_ref, hw0_ref, hw1_ref):
    bf = jnp.bfloat16
    h1 = jnp.maximum(
        jnp.dot(x_ref[...].astype(bf), w1_ref[...],
                preferred_element_type=jnp.float32) + b1_ref[...], 0.0)
    h2 = jnp.dot(h1.astype(bf), w2_ref[...],
                 preferred_element_type=jnp.float32) + b2_ref[...]
    h_ref[...] = h2
    hw = jnp.dot(h2.astype(bf), g_ref[...], preferred_element_type=jnp.float32)
    hw0_ref[...] = hw[:, :_HH]
    hw1_ref[...] = hw[:, _HH:]


def _gru_gates(m0_ref, m1_ref, h_ref, wih_ref, whh_ref, bih_ref, bhh_ref):
    bf = jnp.bfloat16
    m = jnp.concatenate([m0_ref[...], m1_ref[...]], axis=1).astype(bf)
    h = h_ref[...]
    gi = lax.dot_general(m, wih_ref[...], (((1,), (1,)), ((), ())),
                         preferred_element_type=jnp.float32) + bih_ref[...]
    gh = lax.dot_general(h.astype(bf), whh_ref[...], (((1,), (1,)), ((), ())),
                         preferred_element_type=jnp.float32) + bhh_ref[...]
    r = jax.nn.sigmoid(gi[:, :_H] + gh[:, :_H])
    z = jax.nn.sigmoid(gi[:, _H:2 * _H] + gh[:, _H:2 * _H])
    n = jnp.tanh(gi[:, 2 * _H:] + r * gh[:, 2 * _H:])
    return (1.0 - z) * n + z * h


def _gru_mid_body(m0_ref, m1_ref, h_ref, wih_ref, whh_ref, bih_ref, bhh_ref,
                  g_ref, hout_ref, hw0_ref, hw1_ref):
    hn = _gru_gates(m0_ref, m1_ref, h_ref, wih_ref, whh_ref, bih_ref, bhh_ref)
    hout_ref[...] = hn
    hw = jnp.dot(hn.astype(jnp.bfloat16), g_ref[...],
                 preferred_element_type=jnp.float32)
    hw0_ref[...] = hw[:, :_HH]
    hw1_ref[...] = hw[:, _HH:]


def _gru_final_body(m0_ref, m1_ref, h_ref, wih_ref, whh_ref, bih_ref, bhh_ref,
                    dw1_ref, db1_ref, dw2_ref, db2_ref, out_ref):
    hn = _gru_gates(m0_ref, m1_ref, h_ref, wih_ref, whh_ref, bih_ref, bhh_ref)
    y = jnp.maximum(hn, 0.0)
    o = jnp.maximum(
        jnp.dot(y.astype(jnp.bfloat16), dw1_ref[...],
                preferred_element_type=jnp.float32) + db1_ref[...], 0.0)
    o2 = jnp.dot(o.astype(jnp.bfloat16), dw2_ref[...],
                 preferred_element_type=jnp.float32) + db2_ref[...]
    out_ref[...] = jax.nn.sigmoid(o2)


_SUP = 20


def _make_segsum(n_nodes, n_edges):
    ep = n_edges // _NS
    nch = ep // _K
    nsup = nch // _SUP
    rz = (n_nodes // _NS) // 8 * 8
    rrem = n_nodes - rz * _NS
    mesh = plsc.VectorSubcoreMesh(core_axis_name="c", subcore_axis_name="s",
                                  num_cores=_NC, num_subcores=_NS)

    @functools.partial(
        pl.kernel,
        out_type=jax.ShapeDtypeStruct((_NC, n_nodes, _HH), jnp.float32),
        mesh=mesh,
        scratch_types=[
            pltpu.VMEM((_SUP, _K), jnp.int32),
            pltpu.VMEM((_SUP, _K), jnp.int32),
            pltpu.VMEM((_SUP, _K), jnp.int32),
            pltpu.VMEM((_SUP, _K), jnp.int32),
            pltpu.VMEM((_K, _HH), jnp.float32),
            pltpu.VMEM((_K, _HH), jnp.float32),
            pltpu.VMEM_SHARED((n_nodes, _HH), jnp.float32),
            pltpu.SemaphoreType.DMA,
            pltpu.SemaphoreType.DMA,
            pltpu.SemaphoreType.DMA,
            pltpu.SemaphoreType.DMA,
            pltpu.SemaphoreType.DMA,
            pltpu.SemaphoreType.DMA,
        ],
    )
    def segsum(hw, src4, dst4, zblk, m,
               src0, dst0, src1, dst1, rows0, rows1, acc,
               sem0, sem1, ssem0, ssem1, isem0, isem1):
        c = lax.axis_index("c")
        s = lax.axis_index("s")
        rowsb = (rows0, rows1)
        sems = (sem0, sem1)
        srcb = (src0, src1)
        dstb = (dst0, dst1)
        isems = (isem0, isem1)
        srow = (c * _NS + s) * nsup
        drow = s * nsup

        pltpu.sync_copy(zblk, acc.at[pl.ds(s * rz, rz)])
        if rrem:
            @pl.when(s == _NS - 1)
            def _():
                pltpu.sync_copy(zblk.at[pl.ds(0, rrem)],
                                acc.at[pl.ds(_NS * rz, rrem)])
        plsc.subcore_barrier()

        def load_idx(u, p):
            pltpu.async_copy(src4.at[srow + u], srcb[p], isems[p])
            pltpu.async_copy(dst4.at[drow + u], dstb[p], isems[p])

        def wait_idx(u, p):
            pltpu.make_async_copy(src4.at[srow + u], srcb[p], isems[p]).wait()
            pltpu.make_async_copy(dst4.at[drow + u], dstb[p], isems[p]).wait()

        def issue_gather(p, j, b):
            pltpu.async_copy(hw.at[srcb[p].at[j]], rowsb[b], sems[b])

        def wait_gather(p, j, b):
            pltpu.make_async_copy(hw.at[srcb[p].at[j]], rowsb[b], sems[b]).wait()

        load_idx(0, 0)
        load_idx(1, 1)
        wait_idx(0, 0)
        for b in range(2):
            issue_gather(0, b, b)

        def sup_pair(t, carry):
            for p in range(2):
                u = 2 * t + p
                q = 1 - p
                @pl.when(u + 1 < nsup)
                def _():
                    wait_idx(u + 1, q)
                for j in range(_SUP):
                    b = (p * _SUP + j) % 2
                    wait_gather(p, j, b)
                    pltpu.sync_copy(rowsb[b], acc.at[dstb[p].at[j]], add=True)
                    if j + 2 < _SUP:
                        issue_gather(p, j + 2, b)
                    else:
                        @pl.when(u + 1 < nsup)
                        def _():
                            issue_gather(q, j + 2 - _SUP, b)
                @pl.when(u + 2 < nsup)
                def _():
                    load_idx(u + 2, p)
            return carry

        lax.fori_loop(0, nsup // 2, sup_pair, 0)
        plsc.subcore_barrier()
        pltpu.sync_copy(acc.at[pl.ds(s * rz, rz)],
                        m.at[c, pl.ds(s * rz, rz)])
        if rrem:
            @pl.when(s == _NS - 1)
            def _():
                pltpu.sync_copy(acc.at[pl.ds(_NS * rz, rrem)],
                                m.at[c, pl.ds(_NS * rz, rrem)])

    return segsum


def kernel(x, edge_index, enc_w1, enc_b1, enc_w2, enc_b2, ggc_w,
           w_ih, w_hh, b_ih, b_hh, dec_w1, dec_b1, dec_w2, dec_b2):
    n_nodes, d_in = x.shape
    h_dim = enc_w1.shape[1]
    n_layers = ggc_w.shape[0]
    n_edges = edge_index.shape[1]
    grid = (n_nodes // _BN,)

    nsup = (n_edges // _NS) // _K // _SUP
    src_half = edge_index[0].reshape(_NS * nsup, _SUP, _K)
    src2 = jnp.concatenate([src_half, src_half + n_nodes], axis=0)
    dst2 = edge_index[1].reshape(_NS * nsup, _SUP, _K)
    zblk = jnp.zeros(((n_nodes // _NS) // 8 * 8, _HH), jnp.float32)
    b1 = enc_b1.reshape(1, h_dim)
    b2 = enc_b2.reshape(1, h_dim)
    bih = b_ih.reshape(1, 3 * h_dim)
    bhh = b_hh.reshape(1, 3 * h_dim)
    db1 = dec_b1.reshape(1, h_dim)
    db2 = dec_b2.reshape(1, d_in)
    bf = jnp.bfloat16
    enc_w1 = enc_w1.astype(bf)
    enc_w2 = enc_w2.astype(bf)
    ggc_w = ggc_w.astype(bf)
    w_ih = w_ih.astype(bf)
    w_hh = w_hh.astype(bf)
    dec_w1 = dec_w1.astype(bf)
    dec_w2 = dec_w2.astype(bf)

    enc = pl.pallas_call(
        _enc_body,
        grid=grid,
        in_specs=[
            pl.BlockSpec((_BN, d_in), lambda i: (i, 0)),
            _full_spec((d_in, h_dim)),
            _full_spec((1, h_dim)),
            _full_spec((h_dim, h_dim)),
            _full_spec((1, h_dim)),
            _full_spec((h_dim, h_dim)),
        ],
        out_specs=[
            pl.BlockSpec((_BN, h_dim), lambda i: (i, 0)),
            _half_spec(), _half_spec(),
        ],
        out_shape=[
            jax.ShapeDtypeStruct((n_nodes, h_dim), jnp.float32),
            jax.ShapeDtypeStruct((n_nodes, _HH), jnp.float32),
            jax.ShapeDtypeStruct((n_nodes, _HH), jnp.float32),
        ],
    )
    h, hw0, hw1 = enc(x, enc_w1, b1, enc_w2, b2, ggc_w[0])

    segsum = _make_segsum(n_nodes, n_edges)

    nb = n_nodes // _BN
    gru_common_specs = [
        pl.BlockSpec((_BN, _HH), lambda i: (i, 0)),
        pl.BlockSpec((_BN, _HH), lambda i: (i + nb, 0)),
        pl.BlockSpec((_BN, h_dim), lambda i: (i, 0)),
        _full_spec((3 * h_dim, h_dim)),
        _full_spec((3 * h_dim, h_dim)),
        _full_spec((1, 3 * h_dim)),
        _full_spec((1, 3 * h_dim)),
    ]
    gru_mid = pl.pallas_call(
        _gru_mid_body,
        grid=grid,
        in_specs=gru_common_specs + [_full_spec((h_dim, h_dim))],
        out_specs=[
            pl.BlockSpec((_BN, h_dim), lambda i: (i, 0)),
            _half_spec(), _half_spec(),
        ],
        out_shape=[
            jax.ShapeDtypeStruct((n_nodes, h_dim), jnp.float32),
            jax.ShapeDtypeStruct((n_nodes, _HH), jnp.float32),
            jax.ShapeDtypeStruct((n_nodes, _HH), jnp.float32),
        ],
    )
    gru_final = pl.pallas_call(
        _gru_final_body,
        grid=grid,
        in_specs=gru_common_specs + [
            _full_spec((h_dim, h_dim)),
            _full_spec((1, h_dim)),
            _full_spec((h_dim, d_in)),
            _full_spec((1, d_in)),
        ],
        out_specs=[pl.BlockSpec((_BN, d_in), lambda i: (i, 0))],
        out_shape=[jax.ShapeDtypeStruct((n_nodes, d_in), jnp.float32)],
    )

    for l in range(n_layers):
        hw_st = jnp.concatenate([hw0, hw1], axis=0)
        m = segsum(hw_st, src2, dst2, zblk)
        m = m.reshape(_NC * n_nodes, _HH)
        if l + 1 < n_layers:
            h, hw0, hw1 = gru_mid(m, m, h, w_ih, w_hh, bih, bhh, ggc_w[l + 1])
        else:
            (out,) = gru_final(m, m, h, w_ih, w_hh, bih, bhh,
                               dec_w1, db1, dec_w2, db2)
    return out


# speedup vs baseline: 1.2897x; 1.0119x over previous
import functools

import jax
import jax.numpy as jnp
from jax import lax
from jax.experimental import pallas as pl
from jax.experimental.pallas import tpu as pltpu
from jax.experimental.pallas import tpu_sc as plsc

_NC = 2
_NS = 16
_K = 125

_H = 256
_HH = 128
_BN = 2000


def _half_spec():
    return pl.BlockSpec((_BN, _HH), lambda i: (i, 0))


def _full_spec(shape):
    return pl.BlockSpec(shape, lambda i: (0, 0))


def _enc_body(x_ref, w1_ref, b1_ref, w2_ref, b2_ref, g_ref,
              h_ref, hw0_ref, hw1_ref):
    bf = jnp.bfloat16
    h1 = jnp.maximum(
        jnp.dot(x_ref[...].astype(bf), w1_ref[...],
                preferred_element_type=jnp.float32) + b1_ref[...], 0.0)
    h2 = jnp.dot(h1.astype(bf), w2_ref[...],
                 preferred_element_type=jnp.float32) + b2_ref[...]
    h_ref[...] = h2
    hw = jnp.dot(h2.astype(bf), g_ref[...], preferred_element_type=jnp.float32)
    hw0_ref[...] = hw[:, :_HH]
    hw1_ref[...] = hw[:, _HH:]


def _gru_gates(m0_ref, m1_ref, h_ref, wih_ref, whh_ref, bih_ref, bhh_ref):
    bf = jnp.bfloat16
    m = jnp.concatenate([m0_ref[...], m1_ref[...]], axis=1).astype(bf)
    h = h_ref[...]
    gi = lax.dot_general(m, wih_ref[...], (((1,), (1,)), ((), ())),
                         preferred_element_type=jnp.float32) + bih_ref[...]
    gh = lax.dot_general(h.astype(bf), whh_ref[...], (((1,), (1,)), ((), ())),
                         preferred_element_type=jnp.float32) + bhh_ref[...]
    r = jax.nn.sigmoid(gi[:, :_H] + gh[:, :_H])
    z = jax.nn.sigmoid(gi[:, _H:2 * _H] + gh[:, _H:2 * _H])
    n = jnp.tanh(gi[:, 2 * _H:] + r * gh[:, 2 * _H:])
    return (1.0 - z) * n + z * h


def _gru_mid_body(m0_ref, m1_ref, h_ref, wih_ref, whh_ref, bih_ref, bhh_ref,
                  g_ref, hout_ref, hw0_ref, hw1_ref):
    hn = _gru_gates(m0_ref, m1_ref, h_ref, wih_ref, whh_ref, bih_ref, bhh_ref)
    hout_ref[...] = hn
    hw = jnp.dot(hn.astype(jnp.bfloat16), g_ref[...],
                 preferred_element_type=jnp.float32)
    hw0_ref[...] = hw[:, :_HH]
    hw1_ref[...] = hw[:, _HH:]


def _gru_final_body(m0_ref, m1_ref, h_ref, wih_ref, whh_ref, bih_ref, bhh_ref,
                    dw1_ref, db1_ref, dw2_ref, db2_ref, out_ref):
    hn = _gru_gates(m0_ref, m1_ref, h_ref, wih_ref, whh_ref, bih_ref, bhh_ref)
    y = jnp.maximum(hn, 0.0)
    o = jnp.maximum(
        jnp.dot(y.astype(jnp.bfloat16), dw1_ref[...],
                preferred_element_type=jnp.float32) + db1_ref[...], 0.0)
    o2 = jnp.dot(o.astype(jnp.bfloat16), dw2_ref[...],
                 preferred_element_type=jnp.float32) + db2_ref[...]
    out_ref[...] = jax.nn.sigmoid(o2)


_SUP = 20


def _make_segsum(n_nodes, n_edges):
    ep = n_edges // _NS
    nch = ep // _K
    nsup = nch // _SUP
    rz = (n_nodes // _NS) // 8 * 8
    rrem = n_nodes - rz * _NS
    mesh = plsc.VectorSubcoreMesh(core_axis_name="c", subcore_axis_name="s",
                                  num_cores=_NC, num_subcores=_NS)

    @functools.partial(
        pl.kernel,
        out_type=jax.ShapeDtypeStruct((_NC, n_nodes, _HH), jnp.float32),
        mesh=mesh,
        scratch_types=[
            pltpu.VMEM((_SUP, _K), jnp.int32),
            pltpu.VMEM((_SUP, _K), jnp.int32),
            pltpu.VMEM((_SUP, _K), jnp.int32),
            pltpu.VMEM((_SUP, _K), jnp.int32),
            pltpu.VMEM((_K, _HH), jnp.float32),
            pltpu.VMEM((_K, _HH), jnp.float32),
            pltpu.VMEM_SHARED((n_nodes, _HH), jnp.float32),
            pltpu.SemaphoreType.DMA,
            pltpu.SemaphoreType.DMA,
            pltpu.SemaphoreType.DMA,
            pltpu.SemaphoreType.DMA,
            pltpu.SemaphoreType.DMA,
            pltpu.SemaphoreType.DMA,
        ],
    )
    def segsum(hw, src4, dst4, zblk, m,
               src0, dst0, src1, dst1, rows0, rows1, acc,
               sem0, sem1, ssem0, ssem1, isem0, isem1):
        c = lax.axis_index("c")
        s = lax.axis_index("s")
        rowsb = (rows0, rows1)
        sems = (sem0, sem1)
        srcb = (src0, src1)
        dstb = (dst0, dst1)
        isems = (isem0, isem1)
        srow = (c * _NS + s) * nsup
        drow = s * nsup

        pltpu.sync_copy(zblk, acc.at[pl.ds(s * rz, rz)])
        if rrem:
            @pl.when(s == _NS - 1)
            def _():
                pltpu.sync_copy(zblk.at[pl.ds(0, rrem)],
                                acc.at[pl.ds(_NS * rz, rrem)])
        plsc.subcore_barrier()

        def load_idx(u, p):
            pltpu.async_copy(src4.at[srow + u], srcb[p], isems[p])
            pltpu.async_copy(dst4.at[drow + u], dstb[p], isems[p])

        def wait_idx(u, p):
            pltpu.make_async_copy(src4.at[srow + u], srcb[p], isems[p]).wait()
            pltpu.make_async_copy(dst4.at[drow + u], dstb[p], isems[p]).wait()

        def issue_gather(p, j, b):
            pltpu.async_copy(hw.at[srcb[p].at[j]], rowsb[b], sems[b])

        def wait_gather(p, j, b):
            pltpu.make_async_copy(hw.at[srcb[p].at[j]], rowsb[b], sems[b]).wait()

        load_idx(0, 0)
        load_idx(1, 1)
        wait_idx(0, 0)
        for b in range(2):
            issue_gather(0, b, b)

        def sup_pair(t, carry):
            for p in range(2):
                u = 2 * t + p
                q = 1 - p
                @pl.when(u + 1 < nsup)
                def _():
                    wait_idx(u + 1, q)
                for j in range(_SUP):
                    b = (p * _SUP + j) % 2
                    wait_gather(p, j, b)
                    pltpu.sync_copy(rowsb[b], acc.at[dstb[p].at[j]], add=True)
                    if j + 2 < _SUP:
                        issue_gather(p, j + 2, b)
                    else:
                        @pl.when(u + 1 < nsup)
                        def _():
                            issue_gather(q, j + 2 - _SUP, b)
                @pl.when(u + 2 < nsup)
                def _():
                    load_idx(u + 2, p)
            return carry

        lax.fori_loop(0, nsup // 2, sup_pair, 0)
        plsc.subcore_barrier()
        pltpu.sync_copy(acc.at[pl.ds(s * rz, rz)],
                        m.at[c, pl.ds(s * rz, rz)])
        if rrem:
            @pl.when(s == _NS - 1)
            def _():
                pltpu.sync_copy(acc.at[pl.ds(_NS * rz, rrem)],
                                m.at[c, pl.ds(_NS * rz, rrem)])

    return segsum


def kernel(x, edge_index, enc_w1, enc_b1, enc_w2, enc_b2, ggc_w,
           w_ih, w_hh, b_ih, b_hh, dec_w1, dec_b1, dec_w2, dec_b2):
    n_nodes, d_in = x.shape
    h_dim = enc_w1.shape[1]
    n_layers = ggc_w.shape[0]
    n_edges = edge_index.shape[1]
    grid = (n_nodes // _BN,)

    nsup = (n_edges // _NS) // _K // _SUP
    src_half = edge_index[0].reshape(_NS * nsup, _SUP, _K)
    src2 = jnp.concatenate([src_half, src_half + n_nodes], axis=0)
    dst2 = edge_index[1].reshape(_NS * nsup, _SUP, _K)
    zblk = jnp.zeros(((n_nodes // _NS) // 8 * 8, _HH), jnp.float32)
    b1 = enc_b1.reshape(1, h_dim)
    b2 = enc_b2.reshape(1, h_dim)
    bih = b_ih.reshape(1, 3 * h_dim)
    bhh = b_hh.reshape(1, 3 * h_dim)
    db1 = dec_b1.reshape(1, h_dim)
    db2 = dec_b2.reshape(1, d_in)
    bf = jnp.bfloat16
    enc_w1 = enc_w1.astype(bf)
    enc_w2 = enc_w2.astype(bf)
    ggc_w = ggc_w.astype(bf)
    w_ih = w_ih.astype(bf)
    w_hh = w_hh.astype(bf)
    dec_w1 = dec_w1.astype(bf)
    dec_w2 = dec_w2.astype(bf)

    enc = pl.pallas_call(
        _enc_body,
        grid=grid,
        in_specs=[
            pl.BlockSpec((_BN, d_in), lambda i: (i, 0)),
            _full_spec((d_in, h_dim)),
            _full_spec((1, h_dim)),
            _full_spec((h_dim, h_dim)),
            _full_spec((1, h_dim)),
            _full_spec((h_dim, h_dim)),
        ],
        out_specs=[
            pl.BlockSpec((_BN, h_dim), lambda i: (i, 0)),
            _half_spec(), _half_spec(),
        ],
        out_shape=[
            jax.ShapeDtypeStruct((n_nodes, h_dim), jnp.float32),
            jax.ShapeDtypeStruct((n_nodes, _HH), jnp.float32),
            jax.ShapeDtypeStruct((n_nodes, _HH), jnp.float32),
        ],
    )
    h, hw0, hw1 = enc(x, enc_w1, b1, enc_w2, b2, ggc_w[0])

    segsum = _make_segsum(n_nodes, n_edges)

    nb = n_nodes // _BN
    gru_common_specs = [
        pl.BlockSpec((_BN, _HH), lambda i: (i, 0)),
        pl.BlockSpec((_BN, _HH), lambda i: (i + nb, 0)),
        pl.BlockSpec((_BN, h_dim), lambda i: (i, 0)),
        _full_spec((3 * h_dim, h_dim)),
        _full_spec((3 * h_dim, h_dim)),
        _full_spec((1, 3 * h_dim)),
        _full_spec((1, 3 * h_dim)),
    ]
    gru_mid = pl.pallas_call(
        _gru_mid_body,
        grid=grid,
        in_specs=gru_common_specs + [_full_spec((h_dim, h_dim))],
        out_specs=[
            pl.BlockSpec((_BN, h_dim), lambda i: (i, 0)),
            _half_spec(), _half_spec(),
        ],
        out_shape=[
            jax.ShapeDtypeStruct((n_nodes, h_dim), jnp.float32),
            jax.ShapeDtypeStruct((n_nodes, _HH), jnp.float32),
            jax.ShapeDtypeStruct((n_nodes, _HH), jnp.float32),
        ],
    )
    gru_final = pl.pallas_call(
        _gru_final_body,
        grid=grid,
        in_specs=gru_common_specs + [
            _full_spec((h_dim, h_dim)),
            _full_spec((1, h_dim)),
            _full_spec((h_dim, d_in)),
            _full_spec((1, d_in)),
        ],
        out_specs=[pl.BlockSpec((_BN, d_in), lambda i: (i, 0))],
        out_shape=[jax.ShapeDtypeStruct((n_nodes, d_in), jnp.float32)],
    )

    for l in range(n_layers):
        hw_st = jnp.concatenate([hw0, hw1], axis=0)
        m = segsum(hw_st, src2, dst2, zblk)
        m = m.reshape(_NC * n_nodes, _HH)
        if l + 1 < n_layers:
            h, hw0, hw1 = gru_mid(m, m, h, w_ih, w_hh, bih, bhh, ggc_w[l + 1])
        else:
            (out,) = gru_final(m, m, h, w_ih, w_hh, bih, bhh,
                               dec_w1, db1, dec_w2, db2)
    return out
